# baseline (speedup 1.0000x reference)
import numpy as np
import jax
import jax.numpy as jnp
from jax import lax
from jax.experimental import pallas as pl
from jax.experimental.pallas import tpu as pltpu
from jax.experimental.pallas import tpu_sc as plsc

D_MODEL = 128
MAX_LEN = 512
CHUNK = 128


def _positional_encoding(d_model, max_len):
    pos = np.arange(0, max_len).reshape(-1, 1) / np.power(
        10000.0, np.arange(0, d_model, 2) / d_model)
    pe = np.zeros((max_len, d_model), dtype=np.float32)
    pe[:, 0::2] = np.sin(pos)
    pe[:, 1::2] = np.cos(pos)
    return pe


def _build(B, S, V):
    NC, NS = 2, 16
    NW = NC * NS
    total = B * S
    assert total % (NW * CHUNK) == 0
    per_w = total // NW
    n_chunks = per_w // CHUNK
    assert CHUNK <= S
    pe_rows = S + CHUNK

    NBUF = 2
    SLOTS = 2
    SLOT_OF = [b % SLOTS for b in range(NBUF)]
    n_groups = n_chunks // NBUF
    assert n_chunks % NBUF == 0

    mesh = plsc.VectorSubcoreMesh(core_axis_name="c", subcore_axis_name="s")

    @pl.kernel(
        out_type=jax.ShapeDtypeStruct((total, D_MODEL), jnp.float32),
        mesh=mesh,
        scratch_types=[
            pltpu.VMEM((1, n_chunks, CHUNK), jnp.int32),
            pltpu.VMEM((pe_rows, D_MODEL), jnp.float32),
            pltpu.VMEM_SHARED((NS, SLOTS, CHUNK, D_MODEL), jnp.float32),
        ] + [pltpu.VMEM((CHUNK, D_MODEL), jnp.float32)] * NBUF
          + [pltpu.SemaphoreType.DMA] * (NBUF + 2 * SLOTS),
    )
    def k(table_hbm, idx_hbm, pe_hbm, out_hbm, idx_v, pe_v, spm, *bufs_sems):
        bufs = bufs_sems[:NBUF]
        gsems = bufs_sems[NBUF:2 * NBUF]
        csems = bufs_sems[2 * NBUF:2 * NBUF + SLOTS]
        wsems = bufs_sems[2 * NBUF + SLOTS:]
        cid = lax.axis_index("c")
        sid = lax.axis_index("s")
        wid = sid * NC + cid

        pltpu.sync_copy(idx_hbm.at[pl.ds(wid, 1)], idx_v)

        def gather_start(j, b):
            pltpu.async_copy(table_hbm.at[idx_v.at[0, j]], bufs[b], gsems[b])

        def gather_wait(b):
            pltpu.make_async_copy(
                table_hbm.at[pl.ds(0, CHUNK)], bufs[b], gsems[b]).wait()

        def xbar_start(b):
            s = SLOT_OF[b]
            pltpu.async_copy(bufs[b], spm.at[sid, s], csems[s])

        def xbar_wait(b):
            s = SLOT_OF[b]
            pltpu.make_async_copy(bufs[b], spm.at[sid, s], csems[s]).wait()

        def write_start(j, b):
            s = SLOT_OF[b]
            pltpu.async_copy(
                spm.at[sid, s],
                out_hbm.at[pl.ds(wid * per_w + j * CHUNK, CHUNK)], wsems[s])

        def write_wait_slot(s):
            pltpu.make_async_copy(
                spm.at[sid, s], out_hbm.at[pl.ds(0, CHUNK)], wsems[s]).wait()

        def pe_add(j, b):
            off = lax.rem(j * CHUNK, S)
            rows = bufs[b]

            @plsc.parallel_loop(0, CHUNK, unroll=4)
            def _row(r):
                for c in range(D_MODEL // 16):
                    s = pl.ds(c * 16, 16)
                    rows[r, s] = rows[r, s] + pe_v[off + r, s]

        for b in range(NBUF):
            gather_start(b, b)
        pltpu.sync_copy(pe_hbm, pe_v)

        def phase2(j_prev, b_prev, refill_ok):
            xbar_wait(b_prev)
            write_start(j_prev, b_prev)
            if refill_ok is True:
                gather_start(j_prev + NBUF, b_prev)
            elif refill_ok is not False:
                @pl.when(refill_ok)
                def _refill():
                    gather_start(j_prev + NBUF, b_prev)

        @pl.loop(0, n_groups)
        def _group(kk):
            j0 = kk * NBUF
            for b in range(NBUF):
                j = j0 + b
                gather_wait(b)
                pe_add(j, b)

                if b < SLOTS:
                    @pl.when(kk > 0)
                    def _slot_free():
                        write_wait_slot(SLOT_OF[b])
                else:
                    write_wait_slot(SLOT_OF[b])

                xbar_start(b)

                if b == 0:
                    @pl.when(kk > 0)
                    def _p2():
                        phase2(j - 1, NBUF - 1, True)
                else:
                    phase2(j - 1, b - 1, kk < n_groups - 1)

        phase2(n_chunks - 1, NBUF - 1, False)
        for s in range(SLOTS):
            write_wait_slot(s)

    return k


def kernel(X, table):
    B, S = X.shape
    V, D = table.shape
    assert D == D_MODEL
    pe_np = _positional_encoding(D_MODEL, MAX_LEN)[:S]
    pe = jnp.asarray(np.concatenate([pe_np, pe_np[:CHUNK]], axis=0))
    NW = 32
    idx3d = X.astype(jnp.int32).reshape(NW, -1, CHUNK)
    k = _build(B, S, V)
    out = k(table, idx3d, pe)
    return out.reshape(B, S, D)


# speedup vs baseline: 1.0265x; 1.0265x over previous
import numpy as np
import jax
import jax.numpy as jnp
from jax import lax
from jax.experimental import pallas as pl
from jax.experimental.pallas import tpu as pltpu
from jax.experimental.pallas import tpu_sc as plsc

D_MODEL = 128
MAX_LEN = 512
SPLITS = ((0, 104), (104, 96))


def _positional_encoding(d_model, max_len):
    pos = np.arange(0, max_len).reshape(-1, 1) / np.power(
        10000.0, np.arange(0, d_model, 2) / d_model)
    pe = np.zeros((max_len, d_model), dtype=np.float32)
    pe[:, 0::2] = np.sin(pos)
    pe[:, 1::2] = np.cos(pos)
    return pe


def _build(B, S, V):
    NC, NS = 2, 16
    NW = NC * NS
    total = B * S
    assert B % NW == 0 and S == sum(n for _, n in SPLITS)
    seq_per_w = B // NW

    NBUF = 2
    n_groups = seq_per_w // NBUF
    assert seq_per_w % NBUF == 0

    mesh = plsc.VectorSubcoreMesh(core_axis_name="c", subcore_axis_name="s")

    @pl.kernel(
        out_type=jax.ShapeDtypeStruct((total, D_MODEL), jnp.float32),
        mesh=mesh,
        scratch_types=[
            pltpu.VMEM((seq_per_w, S), jnp.int32),
            pltpu.VMEM((S, D_MODEL), jnp.float32),
        ] + [pltpu.VMEM((S, D_MODEL), jnp.float32)] * NBUF
          + [pltpu.SemaphoreType.DMA] * (2 * NBUF),
        compiler_params=pltpu.CompilerParams(use_tc_tiling_on_sc=False),
    )
    def k(table_hbm, x_hbm, pe_hbm, out_hbm, idx_v, pe_v, *bufs_sems):
        bufs = bufs_sems[:NBUF]
        gsems = bufs_sems[NBUF:2 * NBUF]
        wsems = bufs_sems[2 * NBUF:]
        cid = lax.axis_index("c")
        sid = lax.axis_index("s")
        wid = sid * NC + cid
        seq0 = wid * seq_per_w

        pltpu.sync_copy(x_hbm.at[pl.ds(seq0, seq_per_w)], idx_v)

        def gather_start(r, b):
            for off, n in SPLITS:
                pltpu.async_copy(
                    table_hbm.at[idx_v.at[r, pl.ds(off, n)]],
                    bufs[b].at[pl.ds(off, n)], gsems[b])

        def gather_wait(b):
            pltpu.make_async_copy(
                table_hbm.at[pl.ds(0, S)], bufs[b], gsems[b]).wait()

        def write_start(r, b):
            pltpu.async_copy(
                bufs[b], out_hbm.at[pl.ds((seq0 + r) * S, S)], wsems[b])

        def write_wait(b):
            pltpu.make_async_copy(
                bufs[b], out_hbm.at[pl.ds(0, S)], wsems[b]).wait()

        def pe_add(b):
            rows = bufs[b]

            @plsc.parallel_loop(0, S, unroll=4)
            def _row(r):
                for c in range(D_MODEL // 16):
                    s = pl.ds(c * 16, 16)
                    rows[r, s] = rows[r, s] + pe_v[r, s]

        def refill(r_prev, b_prev, refill_ok):
            if refill_ok is True:
                write_wait(b_prev)
                gather_start(r_prev + NBUF, b_prev)
            elif refill_ok is not False:
                @pl.when(refill_ok)
                def _r():
                    write_wait(b_prev)
                    gather_start(r_prev + NBUF, b_prev)

        for b in range(NBUF):
            gather_start(b, b)
        pltpu.sync_copy(pe_hbm, pe_v)

        @pl.loop(0, n_groups)
        def _group(kk):
            r0 = kk * NBUF
            for b in range(NBUF):
                r = r0 + b
                gather_wait(b)
                pe_add(b)
                write_start(r, b)

                if b == 0:
                    @pl.when(kk > 0)
                    def _p2():
                        refill(r - 1, NBUF - 1, True)
                else:
                    refill(r - 1, b - 1, kk < n_groups - 1)

        for b in range(NBUF):
            write_wait(b)

    return k


def kernel(X, table):
    B, S = X.shape
    V, D = table.shape
    assert D == D_MODEL
    pe = jnp.asarray(_positional_encoding(D_MODEL, MAX_LEN)[:S])
    k = _build(B, S, V)
    out = k(table, X.astype(jnp.int32), pe)
    return out.reshape(B, S, D)


# speedup vs baseline: 1.3593x; 1.3242x over previous
import numpy as np
import jax
import jax.numpy as jnp
from jax import lax
from jax.experimental import pallas as pl
from jax.experimental.pallas import tpu as pltpu
from jax.experimental.pallas import tpu_sc as plsc

D_MODEL = 128
MAX_LEN = 512
CHUNK = 128


def _positional_encoding(d_model, max_len):
    pos = np.arange(0, max_len).reshape(-1, 1) / np.power(
        10000.0, np.arange(0, d_model, 2) / d_model)
    pe = np.zeros((max_len, d_model), dtype=np.float32)
    pe[:, 0::2] = np.sin(pos)
    pe[:, 1::2] = np.cos(pos)
    return pe


def _build(B, S, V):
    NC, NS = 2, 16
    NW = NC * NS
    total = B * S
    assert total % (NW * CHUNK) == 0
    per_w = total // NW
    n_chunks = per_w // CHUNK
    assert CHUNK <= S
    pe_rows = S + CHUNK

    NBUF = 5
    n_groups = n_chunks // NBUF
    assert n_chunks % NBUF == 0

    mesh = plsc.VectorSubcoreMesh(core_axis_name="c", subcore_axis_name="s")

    @pl.kernel(
        out_type=jax.ShapeDtypeStruct((total, D_MODEL), jnp.float32),
        mesh=mesh,
        scratch_types=[
            pltpu.VMEM((1, n_chunks, CHUNK), jnp.int32),
            pltpu.VMEM((pe_rows, D_MODEL), jnp.float32),
        ] + [pltpu.VMEM((CHUNK, D_MODEL), jnp.float32)] * NBUF
          + [pltpu.SemaphoreType.DMA] * (2 * NBUF + 2),
    )
    def k(table_hbm, idx_hbm, pe_hbm, out_hbm, idx_v, pe_v, *bufs_sems):
        bufs = bufs_sems[:NBUF]
        gsems = bufs_sems[NBUF:2 * NBUF]
        wsems = bufs_sems[2 * NBUF:3 * NBUF]
        psem1, psem2 = bufs_sems[3 * NBUF:]
        cid = lax.axis_index("c")
        sid = lax.axis_index("s")
        wid = sid * NC + cid

        pltpu.sync_copy(idx_hbm.at[pl.ds(wid, 1)], idx_v)

        def gather_start(j, b):
            pltpu.async_copy(table_hbm.at[idx_v.at[0, j]], bufs[b], gsems[b])

        def gather_wait(b):
            pltpu.make_async_copy(
                table_hbm.at[pl.ds(0, CHUNK)], bufs[b], gsems[b]).wait()

        def write_start(j, b):
            pltpu.async_copy(
                bufs[b], out_hbm.at[pl.ds(wid * per_w + j * CHUNK, CHUNK)],
                wsems[b])

        def write_wait(b):
            pltpu.make_async_copy(
                bufs[b], out_hbm.at[pl.ds(0, CHUNK)], wsems[b]).wait()

        def pe_add(j, b):
            off = lax.rem(j * CHUNK, S)
            rows = bufs[b]

            @plsc.parallel_loop(0, CHUNK, unroll=8)
            def _row(r):
                for c in range(D_MODEL // 16):
                    s = pl.ds(c * 16, 16)
                    rows[r, s] = rows[r, s] + pe_v[off + r, s]

        gather_start(0, 0)
        pltpu.async_copy(
            pe_hbm.at[pl.ds(0, CHUNK)], pe_v.at[pl.ds(0, CHUNK)], psem1)
        gather_start(1, 1)
        pltpu.async_copy(
            pe_hbm.at[pl.ds(CHUNK, pe_rows - CHUNK)],
            pe_v.at[pl.ds(CHUNK, pe_rows - CHUNK)], psem2)
        for b in range(2, NBUF):
            gather_start(b, b)

        @pl.loop(0, n_groups)
        def _group(kk):
            j0 = kk * NBUF
            for b in range(NBUF):
                j = j0 + b
                gather_wait(b)

                if b < 2:
                    @pl.when(kk == 0)
                    def _pe_ready():
                        if b == 0:
                            pltpu.make_async_copy(
                                pe_hbm.at[pl.ds(0, CHUNK)],
                                pe_v.at[pl.ds(0, CHUNK)], psem1).wait()
                        else:
                            pltpu.make_async_copy(
                                pe_hbm.at[pl.ds(CHUNK, pe_rows - CHUNK)],
                                pe_v.at[pl.ds(CHUNK, pe_rows - CHUNK)],
                                psem2).wait()

                pe_add(j, b)
                write_start(j, b)

                @pl.when(kk < n_groups - 1)
                def _refill():
                    write_wait(b)
                    gather_start(j + NBUF, b)

        for b in range(NBUF):
            write_wait(b)

    return k


def kernel(X, table):
    B, S = X.shape
    V, D = table.shape
    assert D == D_MODEL
    pe_np = _positional_encoding(D_MODEL, MAX_LEN)[:S]
    pe = jnp.asarray(np.concatenate([pe_np, pe_np[:CHUNK]], axis=0))
    NW = 32
    idx3d = X.astype(jnp.int32).reshape(NW, -1, CHUNK)
    k = _build(B, S, V)
    out = k(table, idx3d, pe)
    return out.reshape(B, S, D)


# speedup vs baseline: 1.3629x; 1.0027x over previous
import numpy as np
import jax
import jax.numpy as jnp
from jax import lax
from jax.experimental import pallas as pl
from jax.experimental.pallas import tpu as pltpu
from jax.experimental.pallas import tpu_sc as plsc

D_MODEL = 128
MAX_LEN = 512
CHUNK = 128


def _positional_encoding(d_model, max_len):
    pos = np.arange(0, max_len).reshape(-1, 1) / np.power(
        10000.0, np.arange(0, d_model, 2) / d_model)
    pe = np.zeros((max_len, d_model), dtype=np.float32)
    pe[:, 0::2] = np.sin(pos)
    pe[:, 1::2] = np.cos(pos)
    return pe


def _build(B, S, V):
    NC, NS = 2, 16
    NW = NC * NS
    total = B * S
    assert total % (NW * CHUNK) == 0
    per_w = total // NW
    n_chunks = per_w // CHUNK
    assert CHUNK <= S
    pe_rows = S + CHUNK

    NBUF = 5
    n_groups = n_chunks // NBUF
    assert n_chunks % NBUF == 0

    mesh = plsc.VectorSubcoreMesh(core_axis_name="c", subcore_axis_name="s")

    @pl.kernel(
        out_type=jax.ShapeDtypeStruct((total, D_MODEL), jnp.float32),
        mesh=mesh,
        scratch_types=[
            pltpu.VMEM((1, n_chunks, CHUNK), jnp.int32),
            pltpu.VMEM((pe_rows, D_MODEL), jnp.float32),
        ] + [pltpu.VMEM((CHUNK, D_MODEL), jnp.float32)] * NBUF
          + [pltpu.SemaphoreType.DMA] * (2 * NBUF),
    )
    def k(table_hbm, idx_hbm, pe_hbm, out_hbm, idx_v, pe_v, *bufs_sems):
        bufs = bufs_sems[:NBUF]
        gsems = bufs_sems[NBUF:2 * NBUF]
        wsems = bufs_sems[2 * NBUF:]
        cid = lax.axis_index("c")
        sid = lax.axis_index("s")
        wid = sid * NC + cid

        pltpu.sync_copy(idx_hbm.at[pl.ds(wid, 1)], idx_v)

        def gather_start(j, b):
            pltpu.async_copy(table_hbm.at[idx_v.at[0, j]], bufs[b], gsems[b])

        def gather_wait(b):
            pltpu.make_async_copy(
                table_hbm.at[pl.ds(0, CHUNK)], bufs[b], gsems[b]).wait()

        def write_start(j, b):
            pltpu.async_copy(
                bufs[b], out_hbm.at[pl.ds(wid * per_w + j * CHUNK, CHUNK)],
                wsems[b])

        def write_wait(b):
            pltpu.make_async_copy(
                bufs[b], out_hbm.at[pl.ds(0, CHUNK)], wsems[b]).wait()

        def pe_add(j, b):
            off = lax.rem(j * CHUNK, S)
            rows = bufs[b]

            @plsc.parallel_loop(0, CHUNK, unroll=4)
            def _row(r):
                for c in range(D_MODEL // 16):
                    s = pl.ds(c * 16, 16)
                    rows[r, s] = rows[r, s] + pe_v[off + r, s]

        for b in range(NBUF):
            gather_start(b, b)
        pltpu.sync_copy(pe_hbm, pe_v)

        @pl.loop(0, n_groups)
        def _group(kk):
            j0 = kk * NBUF
            for b in range(NBUF):
                j = j0 + b
                gather_wait(b)
                pe_add(j, b)
                write_start(j, b)

                @pl.when(kk < n_groups - 1)
                def _refill():
                    write_wait(b)
                    gather_start(j + NBUF, b)

        for b in range(NBUF):
            write_wait(b)

    return k


def kernel(X, table):
    B, S = X.shape
    V, D = table.shape
    assert D == D_MODEL
    pe_np = _positional_encoding(D_MODEL, MAX_LEN)[:S]
    pe = jnp.asarray(np.concatenate([pe_np, pe_np[:CHUNK]], axis=0))
    NW = 32
    idx3d = X.astype(jnp.int32).reshape(NW, -1, CHUNK)
    k = _build(B, S, V)
    out = k(table, idx3d, pe)
    return out.reshape(B, S, D)
